# Initial kernel scaffold; baseline (speedup 1.0000x reference)
import jax
import jax.numpy as jnp
from jax.experimental import pallas as pl


def kernel(table, rel_mat, W1, b1, W2, b2, src_idx, dst_idx):
    raise NotImplementedError("write your pallas kernel here")


# speedup vs baseline: 3.4118x; 3.4118x over previous
import functools

import jax
import jax.numpy as jnp
from jax import lax
from jax.experimental import pallas as pl
from jax.experimental.pallas import tpu as pltpu
from jax.experimental.pallas import tpu_sc as plsc

N_NODES = 100000
EMB = 128
N_REL = 6
E_PER_REL = 65536

_NC, _NS = 2, 16
_NW = _NC * _NS
_ROWS_TOTAL = 2 * N_REL * E_PER_REL
_ROWS_PER_W = _ROWS_TOTAL // _NW
_CH = 256
_N_CH = _ROWS_PER_W // _CH


def _gather_rows(table, idx_flat):
    mesh = plsc.VectorSubcoreMesh(core_axis_name="c", subcore_axis_name="s")

    @functools.partial(
        pl.kernel,
        mesh=mesh,
        out_type=jax.ShapeDtypeStruct((_ROWS_TOTAL, EMB), jnp.float32),
        scratch_types=[
            pltpu.VMEM((_CH,), jnp.int32),
            pltpu.VMEM((_CH, EMB), jnp.float32),
            pltpu.SemaphoreType.DMA,
        ],
    )
    def gather_kernel(table_hbm, idx_hbm, out_hbm, idx_v, rows_v, gsem):
        wid = lax.axis_index("s") * _NC + lax.axis_index("c")
        base0 = wid * _ROWS_PER_W

        def body(i, carry):
            base = base0 + i * _CH
            pltpu.sync_copy(idx_hbm.at[pl.ds(base, _CH)], idx_v)
            pltpu.async_copy(table_hbm.at[idx_v], rows_v, gsem).wait()
            pltpu.sync_copy(rows_v, out_hbm.at[pl.ds(base, _CH)])
            return carry

        lax.fori_loop(0, _N_CH, body, 0)

    return gather_kernel(table, idx_flat)


_BE = 4096


def _scores(src_g, dst_g, rel_mat):
    def body(src_ref, dst_ref, rel_ref, out_ref):
        s = jnp.dot(src_ref[0], rel_ref[0], preferred_element_type=jnp.float32)
        out_ref[...] = jnp.sum(s * dst_ref[0], axis=-1).reshape(_BE // 512, 512)

    return pl.pallas_call(
        body,
        grid=(N_REL, E_PER_REL // _BE),
        in_specs=[
            pl.BlockSpec((1, _BE, EMB), lambda r, e: (r, e, 0)),
            pl.BlockSpec((1, _BE, EMB), lambda r, e: (r, e, 0)),
            pl.BlockSpec((1, EMB, EMB), lambda r, e: (r, 0, 0)),
        ],
        out_specs=pl.BlockSpec(
            (_BE // 512, 512),
            lambda r, e: (r * (E_PER_REL // _BE) + e, 0),
        ),
        out_shape=jax.ShapeDtypeStruct(
            (N_REL * E_PER_REL // 512, 512), jnp.float32
        ),
    )(src_g, dst_g, rel_mat)


_RB = 5000


def _graph_embd(table, W1, b1_row, W2_row, b2_s):
    def body(t_ref, w1_ref, b1_ref, w2_ref, b2_ref, out_ref, acc):
        i = pl.program_id(0)

        @pl.when(i == 0)
        def _init():
            acc[...] = jnp.zeros_like(acc)

        acc[...] += jnp.sum(t_ref[...], axis=0, keepdims=True)

        @pl.when(i == pl.num_programs(0) - 1)
        def _fin():
            mean = acc[...] * (1.0 / N_NODES)
            h = jnp.maximum(
                jnp.dot(mean, w1_ref[...], preferred_element_type=jnp.float32)
                + b1_ref[...],
                0.0,
            )
            out_ref[...] = jnp.sum(h * w2_ref[...]).reshape(1, 1) + b2_ref[...]

    return pl.pallas_call(
        body,
        grid=(N_NODES // _RB,),
        in_specs=[
            pl.BlockSpec((_RB, EMB), lambda i: (i, 0)),
            pl.BlockSpec((EMB, EMB // 2), lambda i: (0, 0)),
            pl.BlockSpec((1, EMB // 2), lambda i: (0, 0)),
            pl.BlockSpec((1, EMB // 2), lambda i: (0, 0)),
            pl.BlockSpec((1, 1), lambda i: (0, 0)),
        ],
        out_specs=pl.BlockSpec((1, 1), lambda i: (0, 0)),
        out_shape=jax.ShapeDtypeStruct((1, 1), jnp.float32),
        scratch_shapes=[pltpu.VMEM((1, EMB), jnp.float32)],
    )(table, W1, b1_row, W2_row, b2_s)


def kernel(table, rel_mat, W1, b1, W2, b2, src_idx, dst_idx):
    idx_flat = jnp.concatenate(
        [src_idx.reshape(-1), dst_idx.reshape(-1)]
    ).astype(jnp.int32)
    gathered = _gather_rows(table, idx_flat)
    n_edges = N_REL * E_PER_REL
    src_g = gathered[:n_edges].reshape(N_REL, E_PER_REL, EMB)
    dst_g = gathered[n_edges:].reshape(N_REL, E_PER_REL, EMB)
    scores = _scores(src_g, dst_g, rel_mat)
    g = _graph_embd(
        table,
        W1,
        b1.reshape(1, EMB // 2),
        W2.reshape(1, EMB // 2),
        b2.reshape(1, 1),
    )
    return jnp.concatenate([scores.reshape(-1), g.reshape(-1)])
